# Initial kernel scaffold; baseline (speedup 1.0000x reference)
import jax
import jax.numpy as jnp
from jax.experimental import pallas as pl


def kernel(x, table, W1, b1, W2, b2):
    raise NotImplementedError("write your pallas kernel here")


# speedup vs baseline: 5.7622x; 5.7622x over previous
import functools

import jax
import jax.numpy as jnp
from jax import lax
from jax.experimental import pallas as pl
from jax.experimental.pallas import tpu as pltpu
from jax.experimental.pallas import tpu_sc as plsc

B = 16384
L = 200
D = 20
HID = 128

NC = 2
NS = 16
NW = NC * NS
BPW = B // NW
CB = 16
CHUNKS = BPW // CB
ROWS = CB * L
GW = 128
NG = ROWS // GW
XROWS = B * L // GW
PHW = 80


def _sc_body(x_hbm, table_hbm, out_hbm, idx_v, rows_v, acc_v, sem):
  wid = lax.axis_index("s") * NC + lax.axis_index("c")

  iota = lax.iota(jnp.int32, 16)
  rowoff = [(iota + 16 * k) // D for k in range(5)]
  cols = [(iota + 16 * k) % D for k in range(5)]

  def chunk(ci, _):
    b0 = wid * BPW + ci * CB
    pltpu.sync_copy(x_hbm.at[pl.ds(b0 * L, ROWS)], idx_v)
    copies = []
    for j in range(NG):
      copies.append(
          pltpu.async_copy(table_hbm.at[idx_v.at[pl.ds(j * GW, GW)]],
                           rows_v.at[pl.ds(j * GW, GW)], sem))
    for c in copies:
      c.wait()

    for c in range(CB):
      def step(i, accs):
        base = c * L + i * 4
        out = []
        for k in range(5):
          vals = plsc.load_gather(rows_v, [rowoff[k] + base, cols[k]])
          out.append(accs[k] + vals)
        return tuple(out)

      accs = lax.fori_loop(0, L // 4, step,
                           tuple(jnp.zeros((16,), jnp.float32)
                                 for _ in range(5)))
      for k in range(5):
        acc_v[pl.ds(c * PHW + k * 16, 16)] = accs[k]

    pltpu.sync_copy(acc_v, out_hbm.at[pl.ds(b0 * PHW, CB * PHW)])
    return 0

  lax.fori_loop(0, CHUNKS, chunk, 0)


_sc_pool = functools.partial(
    pl.kernel,
    out_type=jax.ShapeDtypeStruct((B * PHW,), jnp.float32),
    mesh=plsc.VectorSubcoreMesh(core_axis_name="c", subcore_axis_name="s"),
    scratch_types=[
        pltpu.VMEM((ROWS,), jnp.int32),
        pltpu.VMEM((ROWS, D), jnp.float32),
        pltpu.VMEM((CB * PHW,), jnp.float32),
        pltpu.SemaphoreType.DMA,
    ],
    compiler_params=pltpu.CompilerParams(
        use_tc_tiling_on_sc=False, needs_layout_passes=False),
)(_sc_body)


BM = 2048


def _tc_body(x_ref, p_ref, w1_ref, b1_ref, w2_ref, b2_ref, o_ref):
  xb = x_ref[...]
  cnt = jnp.sum((xb != 0).astype(jnp.float32), axis=1, keepdims=True)
  p = p_ref[...]
  pooled = (p[:, 0:20] + p[:, 20:40] + p[:, 40:60] + p[:, 60:80])
  pooled = pooled / jnp.maximum(cnt, 1e-9)
  h = jnp.dot(pooled, w1_ref[...], preferred_element_type=jnp.float32)
  h = jnp.maximum(h + b1_ref[...], 0.0)
  o = jnp.sum(h * w2_ref[...], axis=1, keepdims=True) + b2_ref[...]
  o_ref[...] = jax.nn.sigmoid(o)


def _tc_head(x, part, w1t, b1r, w2, b2):
  return pl.pallas_call(
      _tc_body,
      grid=(B // BM,),
      in_specs=[
          pl.BlockSpec((BM, L), lambda i: (i, 0)),
          pl.BlockSpec((BM, PHW), lambda i: (i, 0)),
          pl.BlockSpec((D, HID), lambda i: (0, 0)),
          pl.BlockSpec((1, HID), lambda i: (0, 0)),
          pl.BlockSpec((1, HID), lambda i: (0, 0)),
          pl.BlockSpec((1, 1), lambda i: (0, 0)),
      ],
      out_specs=pl.BlockSpec((BM, 1), lambda i: (i, 0)),
      out_shape=jax.ShapeDtypeStruct((B, 1), jnp.float32),
  )(x, part, w1t, b1r, w2, b2)


def kernel(x, table, W1, b1, W2, b2):
  part = _sc_pool(x.reshape(B * L), table)
  part = part.reshape(B, PHW)
  out = _tc_head(x, part, W1.T, b1.reshape(1, HID), W2,
                 b2.reshape(1, 1))
  return out.reshape(B)
